# baseline (speedup 1.0000x reference)
import jax
import jax.numpy as jnp
from jax import lax
from jax.experimental import pallas as pl
from jax.experimental.pallas import tpu as pltpu, tpu_sc as plsc

N_USERS = 50000
N_ITEMS = 50000
EMB = 32
N_LAYERS = 3
N = N_USERS + N_ITEMS
E = 1600000

NC = 2
NS = 16
HEMB = EMB // NC
CHUNK = 128
BLK = 8
EBLK = BLK * CHUNK
RING = 4
E_TILE = E // NS
NFULL = E_TILE // EBLK
E_TAIL = E_TILE - NFULL * EBLK
TAIL_CHUNKS = 6
HALF = N // NC
HALF_PAD = 50048
SHIFT = HALF_PAD - HALF
N_PAD = NC * HALF_PAD
OUT_STRIPE = N_PAD // NS
OUT_CHUNK = 136
NQ = OUT_STRIPE // OUT_CHUNK
QBOUND = NQ - 1
BREAL = 88


def _body(uemb_hbm, iemb_hbm, eidx_hbm, eval_hbm,
          work_hbm, split_hbm, accm_hbm,
          acc_sh, eib, evb, cidx, ridx, grows, zb, nbuf, abuf, rbuf,
          gsem, ssem):
    c = lax.axis_index("c")
    s = lax.axis_index("s")
    coff = c * N_PAD
    row0 = s * OUT_STRIPE
    ebase = s * E_TILE
    is_user = s < NS // 2
    src0 = row0 - jnp.where(is_user, 0, HALF_PAD)
    is_bnd = jnp.logical_or(s == NS // 2 - 1, s == NS - 1)

    def _split_chunk(tbl, r0, rows):
        pltpu.sync_copy(tbl.at[pl.ds(src0 + r0, rows)],
                        rbuf.at[pl.ds(0, rows)])

        @pl.when(c == 0)
        def _():
            def _x(i, _):
                abuf[i, pl.ds(0, 16)] = rbuf[i, pl.ds(0, 16)]
                return 0

            lax.fori_loop(0, rows, _x, 0)

        @pl.when(c == 1)
        def _():
            def _x(i, _):
                abuf[i, pl.ds(0, 16)] = rbuf[i, pl.ds(16, 16)]
                return 0

            lax.fori_loop(0, rows, _x, 0)

        pltpu.sync_copy(abuf.at[pl.ds(0, rows)],
                        split_hbm.at[pl.ds(coff + row0 + r0, rows)])

    def _split_all(tbl):
        def _q(q, _):
            _split_chunk(tbl, q * OUT_CHUNK, OUT_CHUNK)
            return 0

        lax.fori_loop(0, NQ - 1, _q, 0)

        @pl.when(is_bnd)
        def _():
            _split_chunk(tbl, QBOUND * OUT_CHUNK, BREAL)

        @pl.when(jnp.logical_not(is_bnd))
        def _():
            _split_chunk(tbl, QBOUND * OUT_CHUNK, OUT_CHUNK)

    @pl.when(is_user)
    def _():
        _split_all(uemb_hbm)

    @pl.when(jnp.logical_not(is_user))
    def _():
        _split_all(iemb_hbm)

    zero16 = jnp.zeros((16,), jnp.float32)

    def _z(i, _):
        zb[i, pl.ds(0, 16)] = zero16
        return 0

    lax.fori_loop(0, OUT_CHUNK, _z, 0)

    def _zq(q, _):
        pltpu.sync_copy(zb, acc_sh.at[pl.ds(row0 + q * OUT_CHUNK, OUT_CHUNK)])
        return 0

    lax.fori_loop(0, NQ, _zq, 0)
    plsc.subcore_barrier()

    for L in range(N_LAYERS):
        gsrc = split_hbm if L == 0 else work_hbm

        def _prep(eoff, p):
            def _k(k, _):
                cv = eib[1, pl.ds(eoff + k * 16, 16)]
                cidx[p, pl.ds(k * 16, 16)] = (
                    cv + jnp.where(cv >= HALF, jnp.int32(SHIFT), 0) + coff)
                rv = eib[0, pl.ds(eoff + k * 16, 16)]
                ridx[p, pl.ds(k * 16, 16)] = (
                    rv + jnp.where(rv >= HALF, jnp.int32(SHIFT), 0))
                return 0

            lax.fori_loop(0, CHUNK // 16, _k, 0)

        def _mul(eoff, p):
            def _g(g, _):
                mv = evb[pl.ds(eoff + g * 16, 16)]
                base = g * 16
                for i in range(16):
                    grows[p, base + i, pl.ds(0, 16)] = (
                        grows[p, base + i, pl.ds(0, 16)] * mv[i])
                return 0

            lax.fori_loop(0, CHUNK // 16, _g, 0)

        def _fire_gather(p):
            return pltpu.async_copy(gsrc.at[cidx.at[p]], grows.at[p], gsem[p])

        def _fire_scatter(p):
            return pltpu.async_copy(grows.at[p], acc_sh.at[ridx.at[p]],
                                    ssem[p], add=True)

        def _sweep(nchunks):
            gd = [None] * RING
            sd = [None] * RING
            for j in range(min(RING - 1, nchunks)):
                _prep(j * CHUNK, j)
                gd[j] = _fire_gather(j)
            for j in range(nchunks):
                p = j % RING
                gd[p].wait()
                _mul(j * CHUNK, p)
                sd[p] = _fire_scatter(p)
                nj = j + RING - 1
                if nj < nchunks:
                    np_ = nj % RING
                    if sd[np_] is not None:
                        sd[np_].wait()
                        sd[np_] = None
                    _prep(nj * CHUNK, np_)
                    gd[np_] = _fire_gather(np_)
            for p in range(RING):
                if sd[p] is not None:
                    sd[p].wait()

        def _outer(b, _):
            e0 = ebase + b * EBLK
            pltpu.sync_copy(eidx_hbm.at[0, pl.ds(e0, EBLK)], eib.at[0])
            pltpu.sync_copy(eidx_hbm.at[1, pl.ds(e0, EBLK)], eib.at[1])
            pltpu.sync_copy(eval_hbm.at[pl.ds(e0, EBLK)], evb)
            _sweep(BLK)
            return 0

        lax.fori_loop(0, NFULL, _outer, 0)

        e0 = ebase + NFULL * EBLK
        pltpu.sync_copy(eidx_hbm.at[0, pl.ds(e0, E_TAIL)],
                        eib.at[0, pl.ds(0, E_TAIL)])
        pltpu.sync_copy(eidx_hbm.at[1, pl.ds(e0, E_TAIL)],
                        eib.at[1, pl.ds(0, E_TAIL)])
        pltpu.sync_copy(eval_hbm.at[pl.ds(e0, E_TAIL)],
                        evb.at[pl.ds(0, E_TAIL)])
        for z in range(E_TAIL, TAIL_CHUNKS * CHUNK, 16):
            evb[pl.ds(z, 16)] = zero16
        _sweep(TAIL_CHUNKS)

        plsc.subcore_barrier()

        msrc = split_hbm if L == 0 else accm_hbm
        last = L == N_LAYERS - 1

        def _fold(nrows, r0):
            pltpu.sync_copy(msrc.at[pl.ds(coff + r0, nrows)],
                            abuf.at[pl.ds(0, nrows)])

            def _a(i, _):
                a = abuf[i, pl.ds(0, 16)] + nbuf[i, pl.ds(0, 16)]
                if last:
                    a = a * jnp.float32(1.0 / (N_LAYERS + 1))
                abuf[i, pl.ds(0, 16)] = a
                return 0

            lax.fori_loop(0, nrows, _a, 0)

        def _drain(q, _):
            r0 = row0 + q * OUT_CHUNK
            pltpu.sync_copy(acc_sh.at[pl.ds(r0, OUT_CHUNK)], nbuf)
            if not last:
                pltpu.sync_copy(zb, acc_sh.at[pl.ds(r0, OUT_CHUNK)])
                pltpu.sync_copy(nbuf,
                                work_hbm.at[pl.ds(coff + r0, OUT_CHUNK)])
            _fold(OUT_CHUNK, r0)
            pltpu.sync_copy(abuf,
                            accm_hbm.at[pl.ds(coff + r0, OUT_CHUNK)])
            return 0

        lax.fori_loop(0, NQ, _drain, 0)
        plsc.subcore_barrier()


def kernel(user_emb, item_emb, edge_vals, edge_index):
    mesh = plsc.VectorSubcoreMesh(core_axis_name="c", subcore_axis_name="s",
                                  num_cores=NC, num_subcores=NS)
    outs = pl.kernel(
        _body,
        out_type=(
            jax.ShapeDtypeStruct((NC * N_PAD, HEMB), jnp.float32),
            jax.ShapeDtypeStruct((NC * N_PAD, HEMB), jnp.float32),
            jax.ShapeDtypeStruct((NC * N_PAD, HEMB), jnp.float32),
        ),
        mesh=mesh,
        scratch_types=[
            pltpu.VMEM_SHARED((N_PAD, HEMB), jnp.float32),
            pltpu.VMEM((2, EBLK), jnp.int32),
            pltpu.VMEM((EBLK,), jnp.float32),
            pltpu.VMEM((RING, CHUNK), jnp.int32),
            pltpu.VMEM((RING, CHUNK), jnp.int32),
            pltpu.VMEM((RING, CHUNK, HEMB), jnp.float32),
            pltpu.VMEM((OUT_CHUNK, HEMB), jnp.float32),
            pltpu.VMEM((OUT_CHUNK, HEMB), jnp.float32),
            pltpu.VMEM((OUT_CHUNK, HEMB), jnp.float32),
            pltpu.VMEM((OUT_CHUNK, EMB), jnp.float32),
            tuple(pltpu.SemaphoreType.DMA for _ in range(RING)),
            tuple(pltpu.SemaphoreType.DMA for _ in range(RING)),
        ],
        compiler_params=pltpu.CompilerParams(use_tc_tiling_on_sc=False),
    )(user_emb, item_emb, edge_index, edge_vals)
    accm = outs[2]
    out = (accm.reshape(NC, N_PAD, HEMB)
           .transpose(1, 0, 2)
           .reshape(N_PAD, EMB))
    return out[:N_USERS], out[HALF_PAD:HALF_PAD + N_ITEMS]


# speedup vs baseline: 1.4127x; 1.4127x over previous
import jax
import jax.numpy as jnp
from jax import lax
from jax.experimental import pallas as pl
from jax.experimental.pallas import tpu as pltpu, tpu_sc as plsc

N_USERS = 50000
N_ITEMS = 50000
EMB = 32
N_LAYERS = 3
N = N_USERS + N_ITEMS
E = 1600000

NC = 2
NS = 16
HEMB = EMB // NC
CHUNK = 128
BLK = 16
RING = 6
ROWS_PER_TILE = 784
ROWS = ROWS_PER_TILE * NS
E_PAD = ROWS * CHUNK
HALF = N // NC
HALF_PAD = 50048
N_PAD = NC * HALF_PAD
OUT_STRIPE = N_PAD // NS
OUT_CHUNK = 184


def _body(emb_hbm, edge_hbm, work_hbm, accm_hbm,
          acc_sh, edgb, cidx, grows, zb, nbuf, abuf, gsem, ssem):
    c = lax.axis_index("c")
    s = lax.axis_index("s")
    coff = c * N_PAD
    row0 = s * ROWS_PER_TILE
    out0 = s * OUT_STRIPE

    zero16 = jnp.zeros((16,), jnp.float32)

    def _z(i, _):
        zb[i, pl.ds(0, 16)] = zero16
        return 0

    lax.fori_loop(0, OUT_CHUNK, _z, 0)
    for q in range(OUT_STRIPE // OUT_CHUNK):
        pltpu.sync_copy(zb, acc_sh.at[pl.ds(out0 + q * OUT_CHUNK, OUT_CHUNK)])
    plsc.subcore_barrier()

    for L in range(N_LAYERS):
        gsrc = emb_hbm if L == 0 else work_hbm

        def _outer(b, _):
            rbase = row0 + b * BLK
            pltpu.sync_copy(edge_hbm.at[pl.ds(rbase, BLK)], edgb)

            def _prep(j, p):
                def _k(k, _):
                    cidx[p, pl.ds(k * 16, 16)] = (
                        edgb[j, 0, pl.ds(k * 16, 16)] + coff)
                    return 0

                lax.fori_loop(0, CHUNK // 16, _k, 0)

            def _fire_gather(j, p):
                return pltpu.async_copy(gsrc.at[cidx.at[p]],
                                        grows.at[p], gsem[p])

            gd = [None] * RING
            sd = [None] * RING
            for j in range(RING - 1):
                _prep(j, j)
                gd[j] = _fire_gather(j, j)
            for j in range(BLK):
                p = j % RING
                gd[p].wait()

                def _mul(g, _):
                    mv = plsc.bitcast(edgb[j, 2, pl.ds(g * 16, 16)],
                                      jnp.float32)
                    base = g * 16
                    for i in range(16):
                        grows[p, base + i, pl.ds(0, 16)] = (
                            grows[p, base + i, pl.ds(0, 16)] * mv[i])
                    return 0

                lax.fori_loop(0, CHUNK // 16, _mul, 0)

                sd[p] = pltpu.async_copy(grows.at[p],
                                         acc_sh.at[edgb.at[j, 1]],
                                         ssem[p], add=True)

                nj = j + RING - 1
                if nj < BLK:
                    np_ = nj % RING
                    _prep(nj, np_)
                    if sd[np_] is not None:
                        sd[np_].wait()
                        sd[np_] = None
                    gd[np_] = _fire_gather(nj, np_)
            for p in range(RING):
                if sd[p] is not None:
                    sd[p].wait()
            return 0

        lax.fori_loop(0, ROWS_PER_TILE // BLK, _outer, 0)
        plsc.subcore_barrier()

        msrc = emb_hbm if L == 0 else accm_hbm
        last = L == N_LAYERS - 1
        for q in range(OUT_STRIPE // OUT_CHUNK):
            r0 = out0 + q * OUT_CHUNK
            pltpu.sync_copy(acc_sh.at[pl.ds(r0, OUT_CHUNK)], nbuf)
            if not last:
                pltpu.sync_copy(zb, acc_sh.at[pl.ds(r0, OUT_CHUNK)])
                pltpu.sync_copy(nbuf, work_hbm.at[pl.ds(coff + r0, OUT_CHUNK)])
            pltpu.sync_copy(msrc.at[pl.ds(coff + r0, OUT_CHUNK)], abuf)

            def _acc(i, _):
                a = abuf[i, pl.ds(0, 16)] + nbuf[i, pl.ds(0, 16)]
                if last:
                    a = a * jnp.float32(1.0 / (N_LAYERS + 1))
                abuf[i, pl.ds(0, 16)] = a
                return 0

            lax.fori_loop(0, OUT_CHUNK, _acc, 0)
            pltpu.sync_copy(abuf, accm_hbm.at[pl.ds(coff + r0, OUT_CHUNK)])
        plsc.subcore_barrier()


def _propagate(emb_flat, edges):
    mesh = plsc.VectorSubcoreMesh(core_axis_name="c", subcore_axis_name="s",
                                  num_cores=NC, num_subcores=NS)
    return pl.kernel(
        _body,
        out_type=(
            jax.ShapeDtypeStruct((NC * N_PAD, HEMB), jnp.float32),
            jax.ShapeDtypeStruct((NC * N_PAD, HEMB), jnp.float32),
        ),
        mesh=mesh,
        scratch_types=[
            pltpu.VMEM_SHARED((N_PAD, HEMB), jnp.float32),
            pltpu.VMEM((BLK, 3, CHUNK), jnp.int32),
            pltpu.VMEM((RING, CHUNK), jnp.int32),
            pltpu.VMEM((RING, CHUNK, HEMB), jnp.float32),
            pltpu.VMEM((OUT_CHUNK, HEMB), jnp.float32),
            pltpu.VMEM((OUT_CHUNK, HEMB), jnp.float32),
            pltpu.VMEM((OUT_CHUNK, HEMB), jnp.float32),
            tuple(pltpu.SemaphoreType.DMA for _ in range(RING)),
            tuple(pltpu.SemaphoreType.DMA for _ in range(RING)),
        ],
        compiler_params=pltpu.CompilerParams(use_tc_tiling_on_sc=False,
                                             needs_layout_passes=False),
    )(emb_flat, edges)


def kernel(user_emb, item_emb, edge_vals, edge_index):
    padrows = jnp.zeros((HALF_PAD - HALF, EMB), jnp.float32)
    emb = jnp.concatenate([user_emb, padrows, item_emb, padrows], axis=0)
    emb_flat = (emb.reshape(N_PAD, NC, HEMB)
                .transpose(1, 0, 2)
                .reshape(NC * N_PAD, HEMB))
    row = edge_index[0]
    col = edge_index[1]
    shift = jnp.int32(HALF_PAD - HALF)
    col = col + jnp.where(col >= HALF, shift, 0).astype(jnp.int32)
    row = row + jnp.where(row >= HALF, shift, 0).astype(jnp.int32)
    pad = E_PAD - E
    col2d = jnp.pad(col, (0, pad)).reshape(ROWS, 1, CHUNK)
    row2d = jnp.pad(row, (0, pad)).reshape(ROWS, 1, CHUNK)
    val2d = lax.bitcast_convert_type(
        jnp.pad(edge_vals, (0, pad)).reshape(ROWS, 1, CHUNK), jnp.int32)
    edges = jnp.concatenate([col2d, row2d, val2d], axis=1)
    _, accm = _propagate(emb_flat, edges)
    out = (accm.reshape(NC, N_PAD, HEMB)
           .transpose(1, 0, 2)
           .reshape(N_PAD, EMB))
    return out[:N_USERS], out[HALF_PAD:HALF_PAD + N_ITEMS]


# speedup vs baseline: 1.4628x; 1.0355x over previous
import jax
import jax.numpy as jnp
from jax import lax
from jax.experimental import pallas as pl
from jax.experimental.pallas import tpu as pltpu, tpu_sc as plsc

N_USERS = 50000
N_ITEMS = 50000
EMB = 32
N_LAYERS = 3
N = N_USERS + N_ITEMS
E = 1600000

NC = 2
NS = 16
HEMB = EMB // NC
CHUNK = 128
BLK = 16
RING = 8
ROWS_PER_TILE = 784
ROWS = ROWS_PER_TILE * NS
E_PAD = ROWS * CHUNK
HALF = N // NC
HALF_PAD = 50048
N_PAD = NC * HALF_PAD
OUT_STRIPE = N_PAD // NS
OUT_CHUNK = 136


def _body(emb_hbm, edge_hbm, work_hbm, accm_hbm,
          acc_sh, edgb, cidx, grows, zb, nbuf, abuf, gsem, ssem):
    c = lax.axis_index("c")
    s = lax.axis_index("s")
    coff = c * N_PAD
    row0 = s * ROWS_PER_TILE
    out0 = s * OUT_STRIPE

    zero16 = jnp.zeros((16,), jnp.float32)

    def _z(i, _):
        zb[i, pl.ds(0, 16)] = zero16
        return 0

    lax.fori_loop(0, OUT_CHUNK, _z, 0)
    for q in range(OUT_STRIPE // OUT_CHUNK):
        pltpu.sync_copy(zb, acc_sh.at[pl.ds(out0 + q * OUT_CHUNK, OUT_CHUNK)])
    plsc.subcore_barrier()

    for L in range(N_LAYERS):
        gsrc = emb_hbm if L == 0 else work_hbm

        def _outer(b, _):
            rbase = row0 + b * BLK
            pltpu.sync_copy(edge_hbm.at[pl.ds(rbase, BLK)], edgb)

            def _prep(j, p):
                def _k(k, _):
                    cidx[p, pl.ds(k * 16, 16)] = (
                        edgb[j, 0, pl.ds(k * 16, 16)] + coff)
                    return 0

                lax.fori_loop(0, CHUNK // 16, _k, 0)

            def _fire_gather(j, p):
                return pltpu.async_copy(gsrc.at[cidx.at[p]],
                                        grows.at[p], gsem[p])

            gd = [None] * RING
            sd = [None] * RING
            for j in range(RING - 1):
                _prep(j, j)
                gd[j] = _fire_gather(j, j)
            for j in range(BLK):
                p = j % RING
                gd[p].wait()

                def _mul(g, _):
                    mv = plsc.bitcast(edgb[j, 2, pl.ds(g * 16, 16)],
                                      jnp.float32)
                    base = g * 16
                    for i in range(16):
                        grows[p, base + i, pl.ds(0, 16)] = (
                            grows[p, base + i, pl.ds(0, 16)] * mv[i])
                    return 0

                lax.fori_loop(0, CHUNK // 16, _mul, 0)

                sd[p] = pltpu.async_copy(grows.at[p],
                                         acc_sh.at[edgb.at[j, 1]],
                                         ssem[p], add=True)

                nj = j + RING - 1
                if nj < BLK:
                    np_ = nj % RING
                    _prep(nj, np_)
                    if sd[np_] is not None:
                        sd[np_].wait()
                        sd[np_] = None
                    gd[np_] = _fire_gather(nj, np_)
            for p in range(RING):
                if sd[p] is not None:
                    sd[p].wait()
            return 0

        lax.fori_loop(0, ROWS_PER_TILE // BLK, _outer, 0)
        plsc.subcore_barrier()

        msrc = emb_hbm if L == 0 else accm_hbm
        last = L == N_LAYERS - 1
        for q in range(OUT_STRIPE // OUT_CHUNK):
            r0 = out0 + q * OUT_CHUNK
            pltpu.sync_copy(acc_sh.at[pl.ds(r0, OUT_CHUNK)], nbuf)
            if not last:
                pltpu.sync_copy(zb, acc_sh.at[pl.ds(r0, OUT_CHUNK)])
                pltpu.sync_copy(nbuf, work_hbm.at[pl.ds(coff + r0, OUT_CHUNK)])
            pltpu.sync_copy(msrc.at[pl.ds(coff + r0, OUT_CHUNK)], abuf)

            def _acc(i, _):
                a = abuf[i, pl.ds(0, 16)] + nbuf[i, pl.ds(0, 16)]
                if last:
                    a = a * jnp.float32(1.0 / (N_LAYERS + 1))
                abuf[i, pl.ds(0, 16)] = a
                return 0

            lax.fori_loop(0, OUT_CHUNK, _acc, 0)
            pltpu.sync_copy(abuf, accm_hbm.at[pl.ds(coff + r0, OUT_CHUNK)])
        plsc.subcore_barrier()


def _propagate(emb_flat, edges):
    mesh = plsc.VectorSubcoreMesh(core_axis_name="c", subcore_axis_name="s",
                                  num_cores=NC, num_subcores=NS)
    return pl.kernel(
        _body,
        out_type=(
            jax.ShapeDtypeStruct((NC * N_PAD, HEMB), jnp.float32),
            jax.ShapeDtypeStruct((NC * N_PAD, HEMB), jnp.float32),
        ),
        mesh=mesh,
        scratch_types=[
            pltpu.VMEM_SHARED((N_PAD, HEMB), jnp.float32),
            pltpu.VMEM((BLK, 3, CHUNK), jnp.int32),
            pltpu.VMEM((RING, CHUNK), jnp.int32),
            pltpu.VMEM((RING, CHUNK, HEMB), jnp.float32),
            pltpu.VMEM((OUT_CHUNK, HEMB), jnp.float32),
            pltpu.VMEM((OUT_CHUNK, HEMB), jnp.float32),
            pltpu.VMEM((OUT_CHUNK, HEMB), jnp.float32),
            tuple(pltpu.SemaphoreType.DMA for _ in range(RING)),
            tuple(pltpu.SemaphoreType.DMA for _ in range(RING)),
        ],
        compiler_params=pltpu.CompilerParams(use_tc_tiling_on_sc=False,
                                             needs_layout_passes=False),
    )(emb_flat, edges)


def kernel(user_emb, item_emb, edge_vals, edge_index):
    padrows = jnp.zeros((HALF_PAD - HALF, EMB), jnp.float32)
    emb = jnp.concatenate([user_emb, padrows, item_emb, padrows], axis=0)
    emb_flat = (emb.reshape(N_PAD, NC, HEMB)
                .transpose(1, 0, 2)
                .reshape(NC * N_PAD, HEMB))
    row = edge_index[0]
    col = edge_index[1]
    shift = jnp.int32(HALF_PAD - HALF)
    col = col + jnp.where(col >= HALF, shift, 0).astype(jnp.int32)
    row = row + jnp.where(row >= HALF, shift, 0).astype(jnp.int32)
    pad = E_PAD - E
    col2d = jnp.pad(col, (0, pad)).reshape(ROWS, 1, CHUNK)
    row2d = jnp.pad(row, (0, pad)).reshape(ROWS, 1, CHUNK)
    val2d = lax.bitcast_convert_type(
        jnp.pad(edge_vals, (0, pad)).reshape(ROWS, 1, CHUNK), jnp.int32)
    edges = jnp.concatenate([col2d, row2d, val2d], axis=1)
    _, accm = _propagate(emb_flat, edges)
    out = (accm.reshape(NC, N_PAD, HEMB)
           .transpose(1, 0, 2)
           .reshape(N_PAD, EMB))
    return out[:N_USERS], out[HALF_PAD:HALF_PAD + N_ITEMS]
